# Initial kernel scaffold; baseline (speedup 1.0000x reference)
import jax
import jax.numpy as jnp
from jax.experimental import pallas as pl


def kernel(inputs, categories, codebook):
    raise NotImplementedError("write your pallas kernel here")


# speedup vs baseline: 2.3753x; 2.3753x over previous
import functools

import jax
import jax.numpy as jnp
from jax import lax
from jax.experimental import pallas as pl
from jax.experimental.pallas import tpu as pltpu
from jax.experimental.pallas import tpu_sc as plsc

_NUM_EMBEDDINGS = 8192
_D = 64
_BS = 131072
_NC = 2
_NS = 16
_NW = _NC * _NS
_BW = _BS // _NW
_C = 512
_NCHUNK = _BW // _C


def _vq_body(inputs_hbm, cat_hbm, codebook_hbm, out_hbm, partials_hbm,
             idx_v, rows_v, in_v, acc_v, sem_g, sem_i):
  wid = lax.axis_index("s") * _NC + lax.axis_index("c")
  base = wid * _BW

  def chunk_body(ci, acc):
    tok0 = base + ci * _C
    pltpu.sync_copy(cat_hbm.at[pl.ds(tok0, _C)], idx_v)
    g = pltpu.async_copy(codebook_hbm.at[idx_v], rows_v, sem_g)
    i = pltpu.async_copy(inputs_hbm.at[pl.ds(tok0, _C)], in_v, sem_i)
    g.wait()
    i.wait()

    def row_body(r, acc):
      for j in range(_D // 16):
        d = rows_v[r, pl.ds(j * 16, 16)] - in_v[r, pl.ds(j * 16, 16)]
        acc = acc + d * d
      return acc

    acc = lax.fori_loop(0, _C, row_body, acc)
    pltpu.sync_copy(rows_v, out_hbm.at[pl.ds(tok0, _C)])
    return acc

  acc = lax.fori_loop(0, _NCHUNK, chunk_body,
                      jnp.zeros((16,), jnp.float32))
  acc_v[...] = acc
  pltpu.sync_copy(acc_v, partials_hbm.at[wid])


@jax.jit
def kernel(inputs, categories, codebook):
  mesh = plsc.VectorSubcoreMesh(
      core_axis_name="c", subcore_axis_name="s",
      num_cores=_NC, num_subcores=_NS)
  out, partials = pl.kernel(
      _vq_body,
      out_type=(
          jax.ShapeDtypeStruct((_BS, _D), jnp.float32),
          jax.ShapeDtypeStruct((_NW, 16), jnp.float32),
      ),
      mesh=mesh,
      compiler_params=pltpu.CompilerParams(use_tc_tiling_on_sc=False),
      scratch_types=[
          pltpu.VMEM((_C,), jnp.int32),
          pltpu.VMEM((_C, _D), jnp.float32),
          pltpu.VMEM((_C, _D), jnp.float32),
          pltpu.VMEM((16,), jnp.float32),
          pltpu.SemaphoreType.DMA,
          pltpu.SemaphoreType.DMA,
      ],
  )(inputs, categories, codebook)
  loss = 0.5 * (jnp.sum(partials) / (_BS * _D))
  return out, loss


# speedup vs baseline: 2.6268x; 1.1059x over previous
import jax
import jax.numpy as jnp
from jax import lax
from jax.experimental import pallas as pl
from jax.experimental.pallas import tpu as pltpu
from jax.experimental.pallas import tpu_sc as plsc

_NUM_EMBEDDINGS = 8192
_D = 64
_BS = 131072
_NC = 2
_NS = 16
_NW = _NC * _NS
_BW = _BS // _NW
_C = 256
_NCHUNK = _BW // _C
_NBUF = 3


def _vq_body(inputs_hbm, cat_hbm, codebook_hbm, out_hbm, partials_hbm,
             idx_v, rows_v, in_v, acc_v, sem_g, sem_i, sem_o):
  wid = lax.axis_index("s") * _NC + lax.axis_index("c")
  base = wid * _BW

  pltpu.sync_copy(cat_hbm.at[pl.ds(base, _BW)], idx_v)

  def fire(g):
    b = g % _NBUF
    pltpu.async_copy(codebook_hbm.at[idx_v.at[pl.ds(g * _C, _C)]],
                     rows_v.at[b], sem_g.at[b])
    pltpu.async_copy(inputs_hbm.at[pl.ds(base + g * _C, _C)],
                     in_v.at[b], sem_i.at[b])

  def wait_gi(g):
    b = g % _NBUF
    pltpu.make_async_copy(codebook_hbm.at[idx_v.at[pl.ds(g * _C, _C)]],
                          rows_v.at[b], sem_g.at[b]).wait()
    pltpu.make_async_copy(inputs_hbm.at[pl.ds(base + g * _C, _C)],
                          in_v.at[b], sem_i.at[b]).wait()

  def fire_out(g):
    b = g % _NBUF
    pltpu.async_copy(rows_v.at[b], out_hbm.at[pl.ds(base + g * _C, _C)],
                     sem_o.at[b])

  def wait_out(g):
    b = g % _NBUF
    pltpu.make_async_copy(rows_v.at[b], out_hbm.at[pl.ds(base + g * _C, _C)],
                          sem_o.at[b]).wait()

  fire(0)
  fire(1)
  accs = tuple(jnp.zeros((16,), jnp.float32) for _ in range(4))
  for g in range(_NCHUNK):
    b = g % _NBUF
    wait_gi(g)
    if g + 2 < _NCHUNK:
      if g >= 1:
        wait_out(g - 1)
      fire(g + 2)

    rb = rows_v.at[b]
    ib = in_v.at[b]

    def row_body(r, accs, rb=rb, ib=ib):
      a0, a1, a2, a3 = accs
      d0 = rb[r, pl.ds(0, 16)] - ib[r, pl.ds(0, 16)]
      d1 = rb[r, pl.ds(16, 16)] - ib[r, pl.ds(16, 16)]
      d2 = rb[r, pl.ds(32, 16)] - ib[r, pl.ds(32, 16)]
      d3 = rb[r, pl.ds(48, 16)] - ib[r, pl.ds(48, 16)]
      return (a0 + d0 * d0, a1 + d1 * d1, a2 + d2 * d2, a3 + d3 * d3)

    accs = plsc.parallel_loop(0, _C, step=1, unroll=4, carry=accs)(row_body)
    fire_out(g)

  for g in range(_NCHUNK - 3, _NCHUNK):
    wait_out(g)

  a0, a1, a2, a3 = accs
  acc_v[...] = (a0 + a1) + (a2 + a3)
  pltpu.sync_copy(acc_v, partials_hbm.at[wid])


@jax.jit
def kernel(inputs, categories, codebook):
  mesh = plsc.VectorSubcoreMesh(
      core_axis_name="c", subcore_axis_name="s",
      num_cores=_NC, num_subcores=_NS)
  out, partials = pl.kernel(
      _vq_body,
      out_type=(
          jax.ShapeDtypeStruct((_BS, _D), jnp.float32),
          jax.ShapeDtypeStruct((_NW, 16), jnp.float32),
      ),
      mesh=mesh,
      compiler_params=pltpu.CompilerParams(use_tc_tiling_on_sc=False),
      scratch_types=[
          pltpu.VMEM((_BW,), jnp.int32),
          pltpu.VMEM((_NBUF, _C, _D), jnp.float32),
          pltpu.VMEM((_NBUF, _C, _D), jnp.float32),
          pltpu.VMEM((16,), jnp.float32),
          pltpu.SemaphoreType.DMA((_NBUF,)),
          pltpu.SemaphoreType.DMA((_NBUF,)),
          pltpu.SemaphoreType.DMA((_NBUF,)),
      ],
  )(inputs, categories, codebook)
  loss = 0.5 * (jnp.sum(partials) / (_BS * _D))
  return out, loss


# speedup vs baseline: 4.5805x; 1.7438x over previous
import jax
import jax.numpy as jnp
from jax import lax
from jax.experimental import pallas as pl
from jax.experimental.pallas import tpu as pltpu
from jax.experimental.pallas import tpu_sc as plsc

_NUM_EMBEDDINGS = 8192
_D = 64
_BS = 131072
_NC = 2
_NS = 16
_NW = _NC * _NS
_BW = _BS // _NW
_C = 512
_NCHUNK = _BW // _C
_NBUF = 3

_TB = 2048
_NTB = _BS // _TB


def _gather_body(cat_hbm, cb_hbm, q_hbm, idx_v, rows_v, sem_g, sem_o):
  wid = lax.axis_index("s") * _NC + lax.axis_index("c")
  base = wid * _BW

  pltpu.sync_copy(cat_hbm.at[pl.ds(base, _BW)], idx_v)

  def fire(g):
    b = g % _NBUF
    pltpu.async_copy(cb_hbm.at[idx_v.at[pl.ds(g * _C, _C)]],
                     rows_v.at[b], sem_g.at[b])

  def wait_g(g):
    b = g % _NBUF
    pltpu.make_async_copy(cb_hbm.at[idx_v.at[pl.ds(g * _C, _C)]],
                          rows_v.at[b], sem_g.at[b]).wait()

  def fire_out(g):
    b = g % _NBUF
    pltpu.async_copy(rows_v.at[b],
                     q_hbm.at[pl.ds(base + g * _C, _C), pl.ds(0, _D)],
                     sem_o.at[b])

  def wait_out(g):
    b = g % _NBUF
    pltpu.make_async_copy(rows_v.at[b],
                          q_hbm.at[pl.ds(base + g * _C, _C), pl.ds(0, _D)],
                          sem_o.at[b]).wait()

  fire(0)
  fire(1)
  for g in range(_NCHUNK):
    wait_g(g)
    fire_out(g)
    if g + 2 < _NCHUNK:
      if g >= 1:
        wait_out(g - 1)
      fire(g + 2)
  for g in range(_NCHUNK - 3, _NCHUNK):
    wait_out(g)


def _loss_body(q_ref, x_ref, o_ref, loss_ref):
  i = pl.program_id(0)

  @pl.when(i == 0)
  def _():
    loss_ref[0] = 0.0

  qt = lax.transpose(q_ref[:, 0:_D], (1, 0))
  o_ref[...] = qt
  d = qt - x_ref[...]
  loss_ref[0] += jnp.sum(d * d)


@jax.jit
def kernel(inputs, categories, codebook):
  mesh = plsc.VectorSubcoreMesh(
      core_axis_name="c", subcore_axis_name="s",
      num_cores=_NC, num_subcores=_NS)
  q = pl.kernel(
      _gather_body,
      out_type=jax.ShapeDtypeStruct((_BS, 128), jnp.float32),
      mesh=mesh,
      compiler_params=pltpu.CompilerParams(use_tc_tiling_on_sc=False),
      scratch_types=[
          pltpu.VMEM((_BW,), jnp.int32),
          pltpu.VMEM((_NBUF, _C, _D), jnp.float32),
          pltpu.SemaphoreType.DMA((_NBUF,)),
          pltpu.SemaphoreType.DMA((_NBUF,)),
      ],
  )(categories, codebook)

  out_t, tot = pl.pallas_call(
      _loss_body,
      grid=(_NTB,),
      in_specs=[
          pl.BlockSpec((_TB, 128), lambda i: (i, 0)),
          pl.BlockSpec((_D, _TB), lambda i: (0, i)),
      ],
      out_specs=[
          pl.BlockSpec((_D, _TB), lambda i: (0, i)),
          pl.BlockSpec(memory_space=pltpu.SMEM),
      ],
      out_shape=[
          jax.ShapeDtypeStruct((_D, _BS), jnp.float32),
          jax.ShapeDtypeStruct((1,), jnp.float32),
      ],
      compiler_params=pltpu.CompilerParams(
          dimension_semantics=("arbitrary",)),
  )(q, inputs.T)

  loss = 0.5 * (tot[0] / (_BS * _D))
  return out_t.T, loss


# speedup vs baseline: 5.4748x; 1.1952x over previous
import jax
import jax.numpy as jnp
from jax import lax
from jax.experimental import pallas as pl
from jax.experimental.pallas import tpu as pltpu
from jax.experimental.pallas import tpu_sc as plsc

_NUM_EMBEDDINGS = 8192
_D = 64
_BS = 131072
_NC = 2
_NS = 16
_NW = _NC * _NS
_BW = _BS // _NW
_C = 512
_NCHUNK = _BW // _C
_NBUF = 3

_TB = 4096
_NTB = _BS // _TB


def _gather_body(cat_hbm, cb_hbm, q_hbm, idx_v, rows_v, sem_g, sem_o):
  wid = lax.axis_index("s") * _NC + lax.axis_index("c")
  base = wid * _BW

  pltpu.sync_copy(cat_hbm.at[pl.ds(base, _BW)], idx_v)

  def fire(g):
    b = g % _NBUF
    pltpu.async_copy(cb_hbm.at[idx_v.at[pl.ds(g * _C, _C)]],
                     rows_v.at[b], sem_g.at[b])

  def wait_g(g):
    b = g % _NBUF
    pltpu.make_async_copy(cb_hbm.at[idx_v.at[pl.ds(g * _C, _C)]],
                          rows_v.at[b], sem_g.at[b]).wait()

  def fire_out(g):
    b = g % _NBUF
    pltpu.async_copy(rows_v.at[b],
                     q_hbm.at[pl.ds(base + g * _C, _C), pl.ds(0, _D)],
                     sem_o.at[b])

  def wait_out(g):
    b = g % _NBUF
    pltpu.make_async_copy(rows_v.at[b],
                          q_hbm.at[pl.ds(base + g * _C, _C), pl.ds(0, _D)],
                          sem_o.at[b]).wait()

  fire(0)
  fire(1)
  for g in range(_NCHUNK):
    wait_g(g)
    fire_out(g)
    if g + 2 < _NCHUNK:
      if g >= 1:
        wait_out(g - 1)
      fire(g + 2)
  for g in range(_NCHUNK - 3, _NCHUNK):
    wait_out(g)


def _loss_body(q_ref, x_ref, o_ref, loss_ref):
  i = pl.program_id(0)

  @pl.when(i == 0)
  def _():
    loss_ref[0] = 0.0

  qt = lax.transpose(q_ref[:, 0:_D], (1, 0))
  o_ref[...] = qt
  d = qt - x_ref[...]
  loss_ref[0] += jnp.sum(d * d)


@jax.jit
def kernel(inputs, categories, codebook):
  mesh = plsc.VectorSubcoreMesh(
      core_axis_name="c", subcore_axis_name="s",
      num_cores=_NC, num_subcores=_NS)
  q = pl.kernel(
      _gather_body,
      out_type=jax.ShapeDtypeStruct((_BS, 128), jnp.float32),
      mesh=mesh,
      compiler_params=pltpu.CompilerParams(use_tc_tiling_on_sc=False),
      scratch_types=[
          pltpu.VMEM((_BW,), jnp.int32),
          pltpu.VMEM((_NBUF, _C, _D), jnp.float32),
          pltpu.SemaphoreType.DMA((_NBUF,)),
          pltpu.SemaphoreType.DMA((_NBUF,)),
      ],
  )(categories, codebook)

  out_t, tot = pl.pallas_call(
      _loss_body,
      grid=(_NTB,),
      in_specs=[
          pl.BlockSpec((_TB, 128), lambda i: (i, 0)),
          pl.BlockSpec((_D, _TB), lambda i: (0, i)),
      ],
      out_specs=[
          pl.BlockSpec((_D, _TB), lambda i: (0, i)),
          pl.BlockSpec(memory_space=pltpu.SMEM),
      ],
      out_shape=[
          jax.ShapeDtypeStruct((_D, _BS), jnp.float32),
          jax.ShapeDtypeStruct((1,), jnp.float32),
      ],
      compiler_params=pltpu.CompilerParams(
          dimension_semantics=("arbitrary",)),
  )(q, inputs.T)

  loss = 0.5 * (tot[0] / (_BS * _D))
  return out_t.T, loss


# speedup vs baseline: 5.8814x; 1.0743x over previous
import jax
import jax.numpy as jnp
from jax import lax
from jax.experimental import pallas as pl
from jax.experimental.pallas import tpu as pltpu
from jax.experimental.pallas import tpu_sc as plsc

_NUM_EMBEDDINGS = 8192
_D = 64
_BS = 131072
_NC = 2
_NS = 16
_NW = _NC * _NS
_BW = _BS // _NW
_C = 512
_NCHUNK = _BW // _C
_NBUF = 3

_TB = 8192
_NTB = _BS // _TB


def _gather_body(cat_hbm, cb_hbm, q_hbm, idx_v, rows_v, sem_g, sem_o):
  wid = lax.axis_index("s") * _NC + lax.axis_index("c")
  base = wid * _BW

  pltpu.sync_copy(cat_hbm.at[pl.ds(base, _BW)], idx_v)

  def fire(g):
    b = g % _NBUF
    pltpu.async_copy(cb_hbm.at[idx_v.at[pl.ds(g * _C, _C)]],
                     rows_v.at[b], sem_g.at[b])

  def wait_g(g):
    b = g % _NBUF
    pltpu.make_async_copy(cb_hbm.at[idx_v.at[pl.ds(g * _C, _C)]],
                          rows_v.at[b], sem_g.at[b]).wait()

  def fire_out(g):
    b = g % _NBUF
    pltpu.async_copy(rows_v.at[b],
                     q_hbm.at[pl.ds(base + g * _C, _C), pl.ds(0, _D)],
                     sem_o.at[b])

  def wait_out(g):
    b = g % _NBUF
    pltpu.make_async_copy(rows_v.at[b],
                          q_hbm.at[pl.ds(base + g * _C, _C), pl.ds(0, _D)],
                          sem_o.at[b]).wait()

  fire(0)
  fire(1)
  for g in range(_NCHUNK):
    wait_g(g)
    fire_out(g)
    if g + 2 < _NCHUNK:
      if g >= 1:
        wait_out(g - 1)
      fire(g + 2)
  for g in range(_NCHUNK - 3, _NCHUNK):
    wait_out(g)


def _loss_body(q_ref, x_ref, o_ref, loss_ref):
  i = pl.program_id(0)

  @pl.when(i == 0)
  def _():
    loss_ref[0] = 0.0

  qt = lax.transpose(q_ref[:, 0:_D], (1, 0))
  o_ref[...] = qt
  d = qt - x_ref[...]
  loss_ref[0] += jnp.sum(d * d)


@jax.jit
def kernel(inputs, categories, codebook):
  mesh = plsc.VectorSubcoreMesh(
      core_axis_name="c", subcore_axis_name="s",
      num_cores=_NC, num_subcores=_NS)
  q = pl.kernel(
      _gather_body,
      out_type=jax.ShapeDtypeStruct((_BS, 128), jnp.float32),
      mesh=mesh,
      compiler_params=pltpu.CompilerParams(use_tc_tiling_on_sc=False),
      scratch_types=[
          pltpu.VMEM((_BW,), jnp.int32),
          pltpu.VMEM((_NBUF, _C, _D), jnp.float32),
          pltpu.SemaphoreType.DMA((_NBUF,)),
          pltpu.SemaphoreType.DMA((_NBUF,)),
      ],
  )(categories, codebook)

  out_t, tot = pl.pallas_call(
      _loss_body,
      grid=(_NTB,),
      in_specs=[
          pl.BlockSpec((_TB, 128), lambda i: (i, 0)),
          pl.BlockSpec((_D, _TB), lambda i: (0, i)),
      ],
      out_specs=[
          pl.BlockSpec((_D, _TB), lambda i: (0, i)),
          pl.BlockSpec(memory_space=pltpu.SMEM),
      ],
      out_shape=[
          jax.ShapeDtypeStruct((_D, _BS), jnp.float32),
          jax.ShapeDtypeStruct((1,), jnp.float32),
      ],
      compiler_params=pltpu.CompilerParams(
          dimension_semantics=("arbitrary",)),
  )(q, inputs.T)

  loss = 0.5 * (tot[0] / (_BS * _D))
  return out_t.T, loss


# speedup vs baseline: 5.9252x; 1.0074x over previous
import jax
import jax.numpy as jnp
from jax import lax
from jax.experimental import pallas as pl
from jax.experimental.pallas import tpu as pltpu
from jax.experimental.pallas import tpu_sc as plsc

_NUM_EMBEDDINGS = 8192
_D = 64
_BS = 131072
_NC = 2
_NS = 16
_NW = _NC * _NS
_BW = _BS // _NW
_C = 512
_NCHUNK = _BW // _C
_NBUF = 3

_TB = 16384
_NTB = _BS // _TB


def _gather_body(cat_hbm, cb_hbm, q_hbm, idx_v, rows_v, sem_g, sem_o):
  wid = lax.axis_index("s") * _NC + lax.axis_index("c")
  base = wid * _BW

  pltpu.sync_copy(cat_hbm.at[pl.ds(base, _BW)], idx_v)

  def fire(g):
    b = g % _NBUF
    pltpu.async_copy(cb_hbm.at[idx_v.at[pl.ds(g * _C, _C)]],
                     rows_v.at[b], sem_g.at[b])

  def wait_g(g):
    b = g % _NBUF
    pltpu.make_async_copy(cb_hbm.at[idx_v.at[pl.ds(g * _C, _C)]],
                          rows_v.at[b], sem_g.at[b]).wait()

  def fire_out(g):
    b = g % _NBUF
    pltpu.async_copy(rows_v.at[b],
                     q_hbm.at[pl.ds(base + g * _C, _C), pl.ds(0, _D)],
                     sem_o.at[b])

  def wait_out(g):
    b = g % _NBUF
    pltpu.make_async_copy(rows_v.at[b],
                          q_hbm.at[pl.ds(base + g * _C, _C), pl.ds(0, _D)],
                          sem_o.at[b]).wait()

  fire(0)
  fire(1)
  for g in range(_NCHUNK):
    wait_g(g)
    fire_out(g)
    if g + 2 < _NCHUNK:
      if g >= 1:
        wait_out(g - 1)
      fire(g + 2)
  for g in range(_NCHUNK - 3, _NCHUNK):
    wait_out(g)


def _loss_body(q_ref, x_ref, o_ref, loss_ref):
  i = pl.program_id(0)

  @pl.when(i == 0)
  def _():
    loss_ref[0] = 0.0

  qt = lax.transpose(q_ref[:, 0:_D], (1, 0))
  o_ref[...] = qt
  d = qt - x_ref[...]
  loss_ref[0] += jnp.sum(d * d)


@jax.jit
def kernel(inputs, categories, codebook):
  mesh = plsc.VectorSubcoreMesh(
      core_axis_name="c", subcore_axis_name="s",
      num_cores=_NC, num_subcores=_NS)
  q = pl.kernel(
      _gather_body,
      out_type=jax.ShapeDtypeStruct((_BS, 128), jnp.float32),
      mesh=mesh,
      compiler_params=pltpu.CompilerParams(use_tc_tiling_on_sc=False),
      scratch_types=[
          pltpu.VMEM((_BW,), jnp.int32),
          pltpu.VMEM((_NBUF, _C, _D), jnp.float32),
          pltpu.SemaphoreType.DMA((_NBUF,)),
          pltpu.SemaphoreType.DMA((_NBUF,)),
      ],
  )(categories, codebook)

  out_t, tot = pl.pallas_call(
      _loss_body,
      grid=(_NTB,),
      in_specs=[
          pl.BlockSpec((_TB, 128), lambda i: (i, 0)),
          pl.BlockSpec((_D, _TB), lambda i: (0, i)),
      ],
      out_specs=[
          pl.BlockSpec((_D, _TB), lambda i: (0, i)),
          pl.BlockSpec(memory_space=pltpu.SMEM),
      ],
      out_shape=[
          jax.ShapeDtypeStruct((_D, _BS), jnp.float32),
          jax.ShapeDtypeStruct((1,), jnp.float32),
      ],
      compiler_params=pltpu.CompilerParams(
          dimension_semantics=("arbitrary",)),
  )(q, inputs.T)

  loss = 0.5 * (tot[0] / (_BS * _D))
  return out_t.T, loss


# speedup vs baseline: 6.7903x; 1.1460x over previous
import jax
import jax.numpy as jnp
from jax import lax
from jax.experimental import pallas as pl
from jax.experimental.pallas import tpu as pltpu
from jax.experimental.pallas import tpu_sc as plsc

_NUM_EMBEDDINGS = 8192
_D = 64
_BS = 131072
_NC = 2
_NS = 16
_NW = _NC * _NS
_BW = _BS // _NW
_C = 512
_NCHUNK = _BW // _C
_NBUF = 3

_TB = 16384
_NTB = _BS // _TB


def _gather_body(cat_hbm, cb_hbm, q_hbm, idx_v, rows_v, sem_g, sem_o):
  wid = lax.axis_index("s") * _NC + lax.axis_index("c")
  base = wid * _BW

  pltpu.sync_copy(cat_hbm.at[pl.ds(base, _BW)], idx_v)

  def q_dst(g):
    t0 = base + g * _C
    blk = t0 // _TB
    j = t0 % _TB
    half = j // (_TB // 2)
    row0 = blk * (_TB // 2) + j % (_TB // 2)
    return q_hbm.at[pl.ds(row0, _C), pl.ds(half * _D, _D)]

  def fire(g):
    b = g % _NBUF
    pltpu.async_copy(cb_hbm.at[idx_v.at[pl.ds(g * _C, _C)]],
                     rows_v.at[b], sem_g.at[b])

  def wait_g(g):
    b = g % _NBUF
    pltpu.make_async_copy(cb_hbm.at[idx_v.at[pl.ds(g * _C, _C)]],
                          rows_v.at[b], sem_g.at[b]).wait()

  def fire_out(g):
    b = g % _NBUF
    pltpu.async_copy(rows_v.at[b], q_dst(g), sem_o.at[b])

  def wait_out(g):
    b = g % _NBUF
    pltpu.make_async_copy(rows_v.at[b], q_dst(g), sem_o.at[b]).wait()

  fire(0)
  fire(1)
  for g in range(_NCHUNK):
    wait_g(g)
    fire_out(g)
    if g + 2 < _NCHUNK:
      if g >= 1:
        wait_out(g - 1)
      fire(g + 2)
  for g in range(_NCHUNK - 3, _NCHUNK):
    wait_out(g)


def _loss_body(q_ref, x_ref, o_ref, loss_ref):
  i = pl.program_id(0)

  @pl.when(i == 0)
  def _():
    loss_ref[0] = 0.0

  qt = lax.transpose(q_ref[...], (1, 0))
  a = qt[0:_D, :]
  b = qt[_D:2 * _D, :]
  o_ref[:, 0:_TB // 2] = a
  o_ref[:, _TB // 2:_TB] = b
  x = x_ref[...]
  d1 = a - x[:, 0:_TB // 2]
  d2 = b - x[:, _TB // 2:_TB]
  loss_ref[0] += jnp.sum(d1 * d1) + jnp.sum(d2 * d2)


@jax.jit
def kernel(inputs, categories, codebook):
  mesh = plsc.VectorSubcoreMesh(
      core_axis_name="c", subcore_axis_name="s",
      num_cores=_NC, num_subcores=_NS)
  q = pl.kernel(
      _gather_body,
      out_type=jax.ShapeDtypeStruct((_BS // 2, 128), jnp.float32),
      mesh=mesh,
      compiler_params=pltpu.CompilerParams(use_tc_tiling_on_sc=False),
      scratch_types=[
          pltpu.VMEM((_BW,), jnp.int32),
          pltpu.VMEM((_NBUF, _C, _D), jnp.float32),
          pltpu.SemaphoreType.DMA((_NBUF,)),
          pltpu.SemaphoreType.DMA((_NBUF,)),
      ],
  )(categories, codebook)

  out_t, tot = pl.pallas_call(
      _loss_body,
      grid=(_NTB,),
      in_specs=[
          pl.BlockSpec((_TB // 2, 128), lambda i: (i, 0)),
          pl.BlockSpec((_D, _TB), lambda i: (0, i)),
      ],
      out_specs=[
          pl.BlockSpec((_D, _TB), lambda i: (0, i)),
          pl.BlockSpec(memory_space=pltpu.SMEM),
      ],
      out_shape=[
          jax.ShapeDtypeStruct((_D, _BS), jnp.float32),
          jax.ShapeDtypeStruct((1,), jnp.float32),
      ],
      compiler_params=pltpu.CompilerParams(
          dimension_semantics=("arbitrary",)),
  )(q, inputs.T)

  loss = 0.5 * (tot[0] / (_BS * _D))
  return out_t.T, loss


# speedup vs baseline: 6.9718x; 1.0267x over previous
import jax
import jax.numpy as jnp
from jax import lax
from jax.experimental import pallas as pl
from jax.experimental.pallas import tpu as pltpu
from jax.experimental.pallas import tpu_sc as plsc

_NUM_EMBEDDINGS = 8192
_D = 64
_BS = 131072
_NC = 2
_NS = 16
_NW = _NC * _NS
_BW = _BS // _NW
_C = 512
_NCHUNK = _BW // _C
_NBUF = 3

_TB = 32768
_NTB = _BS // _TB


def _gather_body(cat_hbm, cb_hbm, q_hbm, idx_v, rows_v, sem_g, sem_o):
  wid = lax.axis_index("s") * _NC + lax.axis_index("c")
  base = wid * _BW

  pltpu.sync_copy(cat_hbm.at[pl.ds(base, _BW)], idx_v)

  def q_dst(g):
    t0 = base + g * _C
    blk = t0 // _TB
    j = t0 % _TB
    half = j // (_TB // 2)
    row0 = blk * (_TB // 2) + j % (_TB // 2)
    return q_hbm.at[pl.ds(row0, _C), pl.ds(half * _D, _D)]

  def fire(g):
    b = g % _NBUF
    pltpu.async_copy(cb_hbm.at[idx_v.at[pl.ds(g * _C, _C)]],
                     rows_v.at[b], sem_g.at[b])

  def wait_g(g):
    b = g % _NBUF
    pltpu.make_async_copy(cb_hbm.at[idx_v.at[pl.ds(g * _C, _C)]],
                          rows_v.at[b], sem_g.at[b]).wait()

  def fire_out(g):
    b = g % _NBUF
    pltpu.async_copy(rows_v.at[b], q_dst(g), sem_o.at[b])

  def wait_out(g):
    b = g % _NBUF
    pltpu.make_async_copy(rows_v.at[b], q_dst(g), sem_o.at[b]).wait()

  fire(0)
  fire(1)
  for g in range(_NCHUNK):
    wait_g(g)
    fire_out(g)
    if g + 2 < _NCHUNK:
      if g >= 1:
        wait_out(g - 1)
      fire(g + 2)
  for g in range(_NCHUNK - 3, _NCHUNK):
    wait_out(g)


def _loss_body(q_ref, x_ref, o_ref, loss_ref):
  i = pl.program_id(0)

  @pl.when(i == 0)
  def _():
    loss_ref[0] = 0.0

  qt = lax.transpose(q_ref[...], (1, 0))
  a = qt[0:_D, :]
  b = qt[_D:2 * _D, :]
  o_ref[:, 0:_TB // 2] = a
  o_ref[:, _TB // 2:_TB] = b
  x = x_ref[...]
  d1 = a - x[:, 0:_TB // 2]
  d2 = b - x[:, _TB // 2:_TB]
  loss_ref[0] += jnp.sum(d1 * d1) + jnp.sum(d2 * d2)

  @pl.when(i == _NTB - 1)
  def _():
    loss_ref[0] = loss_ref[0] * (0.5 / (_BS * _D))


@jax.jit
def kernel(inputs, categories, codebook):
  mesh = plsc.VectorSubcoreMesh(
      core_axis_name="c", subcore_axis_name="s",
      num_cores=_NC, num_subcores=_NS)
  q = pl.kernel(
      _gather_body,
      out_type=jax.ShapeDtypeStruct((_BS // 2, 128), jnp.float32),
      mesh=mesh,
      compiler_params=pltpu.CompilerParams(use_tc_tiling_on_sc=False),
      scratch_types=[
          pltpu.VMEM((_BW,), jnp.int32),
          pltpu.VMEM((_NBUF, _C, _D), jnp.float32),
          pltpu.SemaphoreType.DMA((_NBUF,)),
          pltpu.SemaphoreType.DMA((_NBUF,)),
      ],
  )(categories, codebook)

  out_t, tot = pl.pallas_call(
      _loss_body,
      grid=(_NTB,),
      in_specs=[
          pl.BlockSpec((_TB // 2, 128), lambda i: (i, 0)),
          pl.BlockSpec((_D, _TB), lambda i: (0, i)),
      ],
      out_specs=[
          pl.BlockSpec((_D, _TB), lambda i: (0, i)),
          pl.BlockSpec(memory_space=pltpu.SMEM),
      ],
      out_shape=[
          jax.ShapeDtypeStruct((_D, _BS), jnp.float32),
          jax.ShapeDtypeStruct((1,), jnp.float32),
      ],
      compiler_params=pltpu.CompilerParams(
          dimension_semantics=("arbitrary",)),
  )(q, inputs.T)

  return out_t.T, tot[0]
